# baseline (speedup 1.0000x reference)
import functools

import jax
import jax.numpy as jnp
from jax import lax
from jax.experimental import pallas as pl
from jax.experimental.pallas import tpu as pltpu
from jax.experimental.pallas import tpu_sc as plsc

SEQ_LEN = 200
BATCH = 4096
EMB_DIM = 32
NW = 32
CSTRIPE = BATCH // NW
RBLK = 8
NITER = SEQ_LEN // RBLK


def _gather_body(idx_hbm, table_hbm, out_hbm, idx_v, rows_v, sem):
    nc = 2
    wid = lax.axis_index("s") * nc + lax.axis_index("c")
    c0 = wid * CSTRIPE

    def body(i, _):
        r0 = i * RBLK
        pltpu.sync_copy(idx_hbm.at[pl.ds(r0, RBLK), pl.ds(c0, CSTRIPE)], idx_v)
        for j in range(RBLK):
            pltpu.async_copy(table_hbm.at[idx_v.at[j]], rows_v.at[j], sem)
        for j in range(RBLK):
            pltpu.make_async_copy(table_hbm.at[idx_v.at[j]], rows_v.at[j], sem).wait()
        pltpu.sync_copy(rows_v, out_hbm.at[pl.ds(r0, RBLK), pl.ds(c0, CSTRIPE), :])
        return _

    lax.fori_loop(0, NITER, body, None)


@jax.jit
def _emb_lookup(idx, table):
    mesh = plsc.VectorSubcoreMesh(core_axis_name="c", subcore_axis_name="s")
    fn = pl.kernel(
        _gather_body,
        out_type=jax.ShapeDtypeStruct((SEQ_LEN, BATCH, EMB_DIM), jnp.float32),
        mesh=mesh,
        scratch_types=[
            pltpu.VMEM((RBLK, CSTRIPE), jnp.int32),
            pltpu.VMEM((RBLK, CSTRIPE, EMB_DIM), jnp.float32),
            pltpu.SemaphoreType.DMA,
        ],
        compiler_params=pltpu.CompilerParams(use_tc_tiling_on_sc=False),
    )
    return fn(idx, table)


def kernel(inp, lengths, table):
    return _emb_lookup(inp, table)
